# Initial kernel scaffold; baseline (speedup 1.0000x reference)
import jax
import jax.numpy as jnp
from jax.experimental import pallas as pl


def kernel(xyz, feature, wn_w1, wn_g1, wn_b1, wn_w2, wn_g2, wn_b2, wn_w3, wn_g3, wn_b3, final_w, final_g, final_b):
    raise NotImplementedError("write your pallas kernel here")


# speedup vs baseline: 6.5284x; 6.5284x over previous
import functools

import jax
import jax.numpy as jnp
from jax import lax
from jax.experimental import pallas as pl
from jax.experimental.pallas import tpu as pltpu
from jax.experimental.pallas import tpu_sc as plsc

B, N, D = 16, 4096, 128
S, K = 256, 32
WC, OUT_C = 64, 64
NR, NL = 32, 128
P = B * S * K


def _fps_body(x_ref, y_ref, z_ref, nx_ref, ny_ref, nz_ref):
    X = x_ref[...]
    Y = y_ref[...]
    Z = z_ref[...]
    flat = (lax.broadcasted_iota(jnp.int32, (1, NR, NL), 1) * NL
            + lax.broadcasted_iota(jnp.int32, (1, NR, NL), 2))
    out_flat = (lax.broadcasted_iota(jnp.int32, (1, 2, NL), 1) * NL
                + lax.broadcasted_iota(jnp.int32, (1, 2, NL), 2))

    def step(i, carry):
        dist, far, ax, ay, az = carry
        m = flat == far
        cx = jnp.sum(jnp.where(m, X, 0.0), axis=(1, 2), keepdims=True)
        cy = jnp.sum(jnp.where(m, Y, 0.0), axis=(1, 2), keepdims=True)
        cz = jnp.sum(jnp.where(m, Z, 0.0), axis=(1, 2), keepdims=True)
        dd = (X - cx) * (X - cx) + (Y - cy) * (Y - cy) + (Z - cz) * (Z - cz)
        dist = jnp.minimum(dist, dd)
        mx = jnp.max(dist, axis=(1, 2), keepdims=True)
        far_n = jnp.min(jnp.where(dist == mx, flat, jnp.int32(N)),
                        axis=(1, 2), keepdims=True)
        sel = out_flat == i
        ax = jnp.where(sel, cx, ax)
        ay = jnp.where(sel, cy, ay)
        az = jnp.where(sel, cz, az)
        return dist, far_n, ax, ay, az

    init = (jnp.full((B, NR, NL), 1e10, jnp.float32),
            jnp.zeros((B, 1, 1), jnp.int32),
            jnp.zeros((B, 2, NL), jnp.float32),
            jnp.zeros((B, 2, NL), jnp.float32),
            jnp.zeros((B, 2, NL), jnp.float32))
    _, _, ax, ay, az = lax.fori_loop(0, S, step, init)
    nx_ref[...] = ax
    ny_ref[...] = ay
    nz_ref[...] = az


def _run_fps(xp, yp, zp):
    return pl.pallas_call(
        _fps_body,
        out_shape=[jax.ShapeDtypeStruct((B, 2, NL), jnp.float32)] * 3,
    )(xp, yp, zp)


def _knn_body(newp_ref, xyzt_ref, idx_ref):
    new_p = newp_ref[0]
    xyzt = xyzt_ref[0]
    pid = pl.program_id(0)
    n2 = jnp.sum(new_p * new_p, axis=1, keepdims=True)
    x2 = jnp.sum(xyzt * xyzt, axis=0, keepdims=True)
    d = n2 + x2 - 2.0 * jnp.dot(new_p, xyzt,
                                preferred_element_type=jnp.float32)
    iota = lax.broadcasted_iota(jnp.int32, (S, N), 1)
    base = pid * N
    for i in range(K):
        m = jnp.min(d, axis=1, keepdims=True)
        c = jnp.min(jnp.where(d == m, iota, jnp.int32(1 << 30)),
                    axis=1, keepdims=True)
        idx_ref[0, :, i:i + 1] = c + base
        d = jnp.where(iota == c, jnp.float32(3e38), d)


def _run_knn(new_pad, xyzt_pad):
    return pl.pallas_call(
        _knn_body,
        grid=(B,),
        in_specs=[
            pl.BlockSpec((1, S, 8), lambda b: (b, 0, 0)),
            pl.BlockSpec((1, 8, N), lambda b: (b, 0, 0)),
        ],
        out_specs=pl.BlockSpec((1, S, K), lambda b: (b, 0, 0)),
        out_shape=jax.ShapeDtypeStruct((B, S, K), jnp.int32),
    )(new_pad, xyzt_pad)


_NW = 32
_RPW = P // _NW
_CH = 128
_NCH = _RPW // _CH


def _sc_gather(feat_flat, pre_flat, idx3):
    info = plsc.get_sparse_core_info()
    nc = info.num_cores

    @functools.partial(
        pl.kernel,
        mesh=plsc.VectorSubcoreMesh(core_axis_name="c", subcore_axis_name="s"),
        out_type=[jax.ShapeDtypeStruct((P, D), jnp.float32),
                  jax.ShapeDtypeStruct((P, 128), jnp.float32)],
        scratch_types=[pltpu.VMEM((_NCH, _CH), jnp.int32),
                       pltpu.VMEM((_CH, D), jnp.float32),
                       pltpu.VMEM((_CH, 128), jnp.float32),
                       pltpu.SemaphoreType.DMA,
                       pltpu.SemaphoreType.DMA],
    )
    def gk(feat_hbm, pre_hbm, idx_hbm, outf_hbm, outp_hbm,
           idx_v, fr_v, pr_v, sem_f, sem_p):
        wid = lax.axis_index("s") * nc + lax.axis_index("c")
        pltpu.sync_copy(idx_hbm.at[wid], idx_v)

        def body(c, carry):
            cp_f = pltpu.async_copy(feat_hbm.at[idx_v.at[c]], fr_v, sem_f)
            cp_p = pltpu.async_copy(pre_hbm.at[idx_v.at[c]], pr_v, sem_p)
            cp_f.wait()
            cp_p.wait()
            base = wid * _RPW + c * _CH
            pltpu.sync_copy(fr_v, outf_hbm.at[pl.ds(base, _CH)])
            pltpu.sync_copy(pr_v, outp_hbm.at[pl.ds(base, _CH)])
            return carry

        lax.fori_loop(0, _NCH, body, 0)

    return gk(feat_flat, pre_flat, idx3)


_SB = 4
_SC = S // _SB


def _conv_body(gx_ref, newp_ref, gf_ref, w1_ref, w2_ref, w3_ref,
               g1_ref, b1_ref, g2_ref, b2_ref, g3_ref, b3_ref,
               nf_ref, wts_ref):
    bf = jnp.bfloat16
    gx3 = gx_ref[0].reshape(_SC, K, 128)
    rel = (gx3 - newp_ref[0][:, None, :]).reshape(_SC * K, 128)
    h = jnp.dot(rel.astype(bf), w1_ref[...].astype(bf),
                preferred_element_type=jnp.float32)
    h = jnp.maximum(h * g1_ref[...] + b1_ref[...], 0.0)
    h = jnp.dot(h.astype(bf), w2_ref[...].astype(bf),
                preferred_element_type=jnp.float32)
    h = jnp.maximum(h * g2_ref[...] + b2_ref[...], 0.0)
    h = jnp.dot(h.astype(bf), w3_ref[...].astype(bf),
                preferred_element_type=jnp.float32)
    wts_ref[...] = jnp.maximum(h * g3_ref[...] + b3_ref[...], 0.0)

    def body(s, carry):
        fb = gf_ref[0, pl.ds(s * K, K), :].astype(bf)
        wb = wts_ref[pl.ds(s * K, K), :].astype(bf)
        nf = lax.dot_general(fb, wb, (((0,), (0,)), ((), ())),
                             preferred_element_type=jnp.float32)
        nf_ref[0, pl.ds(s * D, D), :] = nf
        return carry

    lax.fori_loop(0, _SC, body, 0)


def _run_conv(gx, newp, gf, w1p, w2t, w3t, g1, b1, g2, b2, g3, b3):
    full = lambda r, c: pl.BlockSpec((r, c), lambda b, j: (0, 0))
    return pl.pallas_call(
        _conv_body,
        grid=(B, _SB),
        in_specs=[
            pl.BlockSpec((1, _SC * K, 128), lambda b, j: (b, j, 0)),
            pl.BlockSpec((1, _SC, 128), lambda b, j: (b, j, 0)),
            pl.BlockSpec((1, _SC * K, D), lambda b, j: (b, j, 0)),
            full(128, WC), full(WC, WC), full(WC, WC),
            full(1, WC), full(1, WC), full(1, WC),
            full(1, WC), full(1, WC), full(1, WC),
        ],
        out_specs=pl.BlockSpec((1, _SC * D, WC), lambda b, j: (b, j, 0)),
        out_shape=jax.ShapeDtypeStruct((B, S * D, WC), jnp.float32),
        scratch_shapes=[pltpu.VMEM((_SC * K, WC), jnp.float32)],
    )(gx, newp, gf, w1p, w2t, w3t, g1, b1, g2, b2, g3, b3)


def _final_body(nf_ref, fw_ref, fg_ref, fb_ref, out_ref):
    o = jnp.dot(nf_ref[0].astype(jnp.bfloat16),
                fw_ref[...].astype(jnp.bfloat16),
                preferred_element_type=jnp.float32)
    out_ref[0] = jnp.maximum(o * fg_ref[...] + fb_ref[...], 0.0)


def _run_final(nfr, fwt, fg, fb):
    return pl.pallas_call(
        _final_body,
        grid=(B,),
        in_specs=[
            pl.BlockSpec((1, S, D * WC), lambda b: (b, 0, 0)),
            pl.BlockSpec((D * WC, OUT_C), lambda b: (0, 0)),
            pl.BlockSpec((1, OUT_C), lambda b: (0, 0)),
            pl.BlockSpec((1, OUT_C), lambda b: (0, 0)),
        ],
        out_specs=pl.BlockSpec((1, S, OUT_C), lambda b: (b, 0, 0)),
        out_shape=jax.ShapeDtypeStruct((B, S, OUT_C), jnp.float32),
    )(nfr, fwt, fg, fb)


def kernel(xyz, feature, wn_w1, wn_g1, wn_b1, wn_w2, wn_g2, wn_b2,
           wn_w3, wn_g3, wn_b3, final_w, final_g, final_b):
    xt = jnp.transpose(xyz, (2, 0, 1)).reshape(3, B, NR, NL)
    nx, ny, nz = _run_fps(xt[0], xt[1], xt[2])

    new_xyz = jnp.stack(
        [nx.reshape(B, S), ny.reshape(B, S), nz.reshape(B, S)], axis=-1)
    new_pad = jnp.pad(new_xyz, ((0, 0), (0, 0), (0, 5)))
    xyzt_pad = jnp.pad(jnp.transpose(xyz, (0, 2, 1)),
                       ((0, 0), (0, 5), (0, 0)))
    idx = _run_knn(new_pad, xyzt_pad)

    feat_flat = jnp.transpose(feature, (0, 2, 1)).reshape(B * N, D)
    xyzp_flat = jnp.pad(xyz, ((0, 0), (0, 0), (0, 125))).reshape(B * N, 128)
    idx3 = idx.reshape(_NW, _NCH, _CH)
    gfeat, gxyz = _sc_gather(feat_flat, xyzp_flat, idx3)

    gx = gxyz.reshape(B, S * K, 128)
    gf = gfeat.reshape(B, S * K, D)
    newp128 = jnp.pad(new_xyz, ((0, 0), (0, 0), (0, 125)))
    w1p = jnp.pad(wn_w1, ((0, 0), (0, 125))).T

    nfbig = _run_conv(gx, newp128, gf, w1p, wn_w2.T, wn_w3.T,
                      wn_g1[None, :], wn_b1[None, :],
                      wn_g2[None, :], wn_b2[None, :],
                      wn_g3[None, :], wn_b3[None, :])

    nfr = nfbig.reshape(B, S, D * WC)
    out_so = _run_final(nfr, final_w.T,
                        final_g[None, :], final_b[None, :])
    out = jnp.transpose(out_so, (0, 2, 1))
    return new_xyz, out


# speedup vs baseline: 9.9895x; 1.5301x over previous
import functools

import jax
import jax.numpy as jnp
from jax import lax
from jax.experimental import pallas as pl
from jax.experimental.pallas import tpu as pltpu
from jax.experimental.pallas import tpu_sc as plsc

B, N, D = 16, 4096, 128
S, K = 256, 32
WC, OUT_C = 64, 64
NR, NL = 32, 128
P = B * S * K


def _fps_body(x_ref, y_ref, z_ref, nx_ref, ny_ref, nz_ref):
    X = x_ref[...]
    Y = y_ref[...]
    Z = z_ref[...]
    flat = (lax.broadcasted_iota(jnp.int32, (1, NR, NL), 1) * NL
            + lax.broadcasted_iota(jnp.int32, (1, NR, NL), 2))
    out_flat = (lax.broadcasted_iota(jnp.int32, (1, 2, NL), 1) * NL
                + lax.broadcasted_iota(jnp.int32, (1, 2, NL), 2))

    def step(i, carry):
        dist, far, ax, ay, az = carry
        m = flat == far
        cx = jnp.sum(jnp.where(m, X, 0.0), axis=(1, 2), keepdims=True)
        cy = jnp.sum(jnp.where(m, Y, 0.0), axis=(1, 2), keepdims=True)
        cz = jnp.sum(jnp.where(m, Z, 0.0), axis=(1, 2), keepdims=True)
        dd = (X - cx) * (X - cx) + (Y - cy) * (Y - cy) + (Z - cz) * (Z - cz)
        dist = jnp.minimum(dist, dd)
        mx = jnp.max(dist, axis=(1, 2), keepdims=True)
        far_n = jnp.min(jnp.where(dist == mx, flat, jnp.int32(N)),
                        axis=(1, 2), keepdims=True)
        sel = out_flat == i
        ax = jnp.where(sel, cx, ax)
        ay = jnp.where(sel, cy, ay)
        az = jnp.where(sel, cz, az)
        return dist, far_n, ax, ay, az

    init = (jnp.full((B, NR, NL), 1e10, jnp.float32),
            jnp.zeros((B, 1, 1), jnp.int32),
            jnp.zeros((B, 2, NL), jnp.float32),
            jnp.zeros((B, 2, NL), jnp.float32),
            jnp.zeros((B, 2, NL), jnp.float32))
    _, _, ax, ay, az = lax.fori_loop(0, S, step, init, unroll=4)
    nx_ref[...] = ax
    ny_ref[...] = ay
    nz_ref[...] = az


def _run_fps(xp, yp, zp):
    return pl.pallas_call(
        _fps_body,
        out_shape=[jax.ShapeDtypeStruct((B, 2, NL), jnp.float32)] * 3,
    )(xp, yp, zp)


def _knn_body(newp_ref, xyzt_ref, idx_ref):
    new_p = newp_ref[0]
    xyzt = xyzt_ref[0]
    pid = pl.program_id(0)
    n2 = jnp.sum(new_p * new_p, axis=1, keepdims=True)
    x2 = jnp.sum(xyzt * xyzt, axis=0, keepdims=True)
    d = n2 + x2 - 2.0 * jnp.dot(new_p, xyzt,
                                preferred_element_type=jnp.float32)
    iota = lax.broadcasted_iota(jnp.int32, (S, N), 1)
    base = pid * N
    for i in range(K):
        m = jnp.min(d, axis=1, keepdims=True)
        c = jnp.min(jnp.where(d == m, iota, jnp.int32(1 << 30)),
                    axis=1, keepdims=True)
        idx_ref[0, :, i:i + 1] = c + base
        d = jnp.where(iota == c, jnp.float32(3e38), d)


def _run_knn(new_pad, xyzt_pad):
    return pl.pallas_call(
        _knn_body,
        grid=(B,),
        in_specs=[
            pl.BlockSpec((1, S, 8), lambda b: (b, 0, 0)),
            pl.BlockSpec((1, 8, N), lambda b: (b, 0, 0)),
        ],
        out_specs=pl.BlockSpec((1, S, K), lambda b: (b, 0, 0)),
        out_shape=jax.ShapeDtypeStruct((B, S, K), jnp.int32),
    )(new_pad, xyzt_pad)


_NW = 32
_RPW = P // _NW
_CH = 128
_NCH = _RPW // _CH


def _sc_gather(feat_flat, pre_flat, idx3):
    info = plsc.get_sparse_core_info()
    nc = info.num_cores

    @functools.partial(
        pl.kernel,
        mesh=plsc.VectorSubcoreMesh(core_axis_name="c", subcore_axis_name="s"),
        out_type=[jax.ShapeDtypeStruct((P, D), jnp.float32),
                  jax.ShapeDtypeStruct((P, 128), jnp.float32)],
        scratch_types=[pltpu.VMEM((_NCH, _CH), jnp.int32),
                       pltpu.VMEM((_CH, D), jnp.float32),
                       pltpu.VMEM((_CH, 128), jnp.float32),
                       pltpu.SemaphoreType.DMA,
                       pltpu.SemaphoreType.DMA],
    )
    def gk(feat_hbm, pre_hbm, idx_hbm, outf_hbm, outp_hbm,
           idx_v, fr_v, pr_v, sem_f, sem_p):
        wid = lax.axis_index("s") * nc + lax.axis_index("c")
        pltpu.sync_copy(idx_hbm.at[wid], idx_v)

        def body(c, carry):
            cp_f = pltpu.async_copy(feat_hbm.at[idx_v.at[c]], fr_v, sem_f)
            cp_p = pltpu.async_copy(pre_hbm.at[idx_v.at[c]], pr_v, sem_p)
            cp_f.wait()
            cp_p.wait()
            base = wid * _RPW + c * _CH
            pltpu.sync_copy(fr_v, outf_hbm.at[pl.ds(base, _CH)])
            pltpu.sync_copy(pr_v, outp_hbm.at[pl.ds(base, _CH)])
            return carry

        lax.fori_loop(0, _NCH, body, 0)

    return gk(feat_flat, pre_flat, idx3)


_SB = 2
_SC = S // _SB


def _conv_body(gx_ref, newp_ref, gf_ref, w1_ref, w2_ref, w3_ref,
               g1_ref, b1_ref, g2_ref, b2_ref, g3_ref, b3_ref,
               nf_ref, wts_ref):
    bf = jnp.bfloat16
    gx3 = gx_ref[0].reshape(_SC, K, 128)
    rel = (gx3 - newp_ref[0][:, None, :]).reshape(_SC * K, 128)
    h = jnp.dot(rel.astype(bf), w1_ref[...].astype(bf),
                preferred_element_type=jnp.float32)
    h = jnp.maximum(h * g1_ref[...] + b1_ref[...], 0.0)
    h = jnp.dot(h.astype(bf), w2_ref[...].astype(bf),
                preferred_element_type=jnp.float32)
    h = jnp.maximum(h * g2_ref[...] + b2_ref[...], 0.0)
    h = jnp.dot(h.astype(bf), w3_ref[...].astype(bf),
                preferred_element_type=jnp.float32)
    wts_ref[...] = jnp.maximum(h * g3_ref[...] + b3_ref[...], 0.0)

    def body(s, carry):
        fb = gf_ref[0, pl.ds(s * K, K), :].astype(bf)
        wb = wts_ref[pl.ds(s * K, K), :].astype(bf)
        nf = lax.dot_general(fb, wb, (((0,), (0,)), ((), ())),
                             preferred_element_type=jnp.float32)
        nf_ref[0, pl.ds(s * D, D), :] = nf
        return carry

    lax.fori_loop(0, _SC, body, 0, unroll=16)


def _run_conv(gx, newp, gf, w1p, w2t, w3t, g1, b1, g2, b2, g3, b3):
    full = lambda r, c: pl.BlockSpec((r, c), lambda b, j: (0, 0))
    return pl.pallas_call(
        _conv_body,
        grid=(B, _SB),
        in_specs=[
            pl.BlockSpec((1, _SC * K, 128), lambda b, j: (b, j, 0)),
            pl.BlockSpec((1, _SC, 128), lambda b, j: (b, j, 0)),
            pl.BlockSpec((1, _SC * K, D), lambda b, j: (b, j, 0)),
            full(128, WC), full(WC, WC), full(WC, WC),
            full(1, WC), full(1, WC), full(1, WC),
            full(1, WC), full(1, WC), full(1, WC),
        ],
        out_specs=pl.BlockSpec((1, _SC * D, WC), lambda b, j: (b, j, 0)),
        out_shape=jax.ShapeDtypeStruct((B, S * D, WC), jnp.float32),
        scratch_shapes=[pltpu.VMEM((_SC * K, WC), jnp.float32)],
    )(gx, newp, gf, w1p, w2t, w3t, g1, b1, g2, b2, g3, b3)


def _final_body(nf_ref, fw_ref, fg_ref, fb_ref, out_ref):
    o = jnp.dot(nf_ref[0].astype(jnp.bfloat16),
                fw_ref[...].astype(jnp.bfloat16),
                preferred_element_type=jnp.float32)
    out_ref[0] = jnp.maximum(o * fg_ref[...] + fb_ref[...], 0.0)


def _run_final(nfr, fwt, fg, fb):
    return pl.pallas_call(
        _final_body,
        grid=(B,),
        in_specs=[
            pl.BlockSpec((1, S, D * WC), lambda b: (b, 0, 0)),
            pl.BlockSpec((D * WC, OUT_C), lambda b: (0, 0)),
            pl.BlockSpec((1, OUT_C), lambda b: (0, 0)),
            pl.BlockSpec((1, OUT_C), lambda b: (0, 0)),
        ],
        out_specs=pl.BlockSpec((1, S, OUT_C), lambda b: (b, 0, 0)),
        out_shape=jax.ShapeDtypeStruct((B, S, OUT_C), jnp.float32),
    )(nfr, fwt, fg, fb)


def kernel(xyz, feature, wn_w1, wn_g1, wn_b1, wn_w2, wn_g2, wn_b2,
           wn_w3, wn_g3, wn_b3, final_w, final_g, final_b):
    xt = jnp.transpose(xyz, (2, 0, 1)).reshape(3, B, NR, NL)
    nx, ny, nz = _run_fps(xt[0], xt[1], xt[2])

    new_xyz = jnp.stack(
        [nx.reshape(B, S), ny.reshape(B, S), nz.reshape(B, S)], axis=-1)
    new_pad = jnp.pad(new_xyz, ((0, 0), (0, 0), (0, 5)))
    xyzt_pad = jnp.pad(jnp.transpose(xyz, (0, 2, 1)),
                       ((0, 0), (0, 5), (0, 0)))
    idx = _run_knn(new_pad, xyzt_pad)

    feat_flat = jnp.transpose(feature, (0, 2, 1)).reshape(B * N, D)
    xyzp_flat = jnp.pad(xyz, ((0, 0), (0, 0), (0, 125))).reshape(B * N, 128)
    idx3 = idx.reshape(_NW, _NCH, _CH)
    gfeat, gxyz = _sc_gather(feat_flat, xyzp_flat, idx3)

    gx = gxyz.reshape(B, S * K, 128)
    gf = gfeat.reshape(B, S * K, D)
    newp128 = jnp.pad(new_xyz, ((0, 0), (0, 0), (0, 125)))
    w1p = jnp.pad(wn_w1, ((0, 0), (0, 125))).T

    nfbig = _run_conv(gx, newp128, gf, w1p, wn_w2.T, wn_w3.T,
                      wn_g1[None, :], wn_b1[None, :],
                      wn_g2[None, :], wn_b2[None, :],
                      wn_g3[None, :], wn_b3[None, :])

    nfr = nfbig.reshape(B, S, D * WC)
    out_so = _run_final(nfr, final_w.T,
                        final_g[None, :], final_b[None, :])
    out = jnp.transpose(out_so, (0, 2, 1))
    return new_xyz, out
